# baseline (speedup 1.0000x reference)
import functools

import jax
import jax.numpy as jnp
from jax import lax
from jax.experimental import pallas as pl
from jax.experimental.pallas import tpu as pltpu
from jax.experimental.pallas import tpu_sc as plsc

N_NODES = 10000
N_EDGES = 320000
D_FEAT = 128
HID = 64

NC = 2
NS = 16
NW = NC * NS
EPW = N_EDGES // NW
CHUNK = 128
NCH = 80
PPW = NCH * CHUNK - EPW
NB = 8
KLA = 4
Y_ROWS = N_NODES + 16
RPT = N_NODES // NS


def _proj_body(x_ref, w1_ref, y_ref):
    i = pl.program_id(0)
    y = lax.dot_general(x_ref[...], w1_ref[...], (((1,), (1,)), ((), ())),
                        preferred_element_type=jnp.float32)
    blk = y.shape[0]
    row = i * blk + lax.broadcasted_iota(jnp.int32, y.shape, 0)
    y_ref[...] = jnp.where(row < N_NODES, y, 0.0)


def _project(x, w1):
    blk = 2504
    return pl.pallas_call(
        _proj_body,
        grid=(Y_ROWS // blk,),
        in_specs=[
            pl.BlockSpec((blk, D_FEAT), lambda i: (i, 0)),
            pl.BlockSpec((HID, D_FEAT), lambda i: (0, 0)),
        ],
        out_specs=pl.BlockSpec((blk, HID), lambda i: (i, 0)),
        out_shape=jax.ShapeDtypeStruct((Y_ROWS, HID), jnp.float32),
    )(x, w1)


def _segsum_body(y_hbm, eidx_hbm, out_hbm,
                 si_flat, di_flat, rows, acc, *gsems):
    c = lax.axis_index("c")
    s = lax.axis_index("s")
    wid = s * NC + c
    base = wid * EPW
    iota = lax.iota(jnp.int32, 16)

    pltpu.sync_copy(eidx_hbm.at[0, pl.ds(base, EPW)],
                    si_flat.at[pl.ds(0, EPW)])
    for j in range(PPW // 16):
        si_flat[pl.ds(EPW + j * 16, 16)] = N_NODES + iota
    for b in range(1, KLA):
        pltpu.make_async_copy(
            y_hbm.at[si_flat.at[pl.ds(b * CHUNK, CHUNK)]],
            rows.at[b], gsems[b]).start()

    pltpu.sync_copy(eidx_hbm.at[1, pl.ds(base, EPW)],
                    di_flat.at[pl.ds(0, EPW)])

    z16 = jnp.zeros((16,), jnp.float32)

    def _zfill(i, carry):
        rows[0, i // 4, pl.ds((i % 4) * 16, 16)] = z16
        return carry

    lax.fori_loop(0, CHUNK * 4, _zfill, 0)
    for k in range(RPT // CHUNK):
        pltpu.sync_copy(rows.at[0], acc.at[pl.ds(s * RPT + k * CHUNK, CHUNK)])
    _zrem = RPT % CHUNK
    if _zrem:
        pltpu.sync_copy(rows.at[0].at[pl.ds(0, _zrem)],
                        acc.at[pl.ds(s * RPT + (RPT // CHUNK) * CHUNK, _zrem)])
    pltpu.make_async_copy(
        y_hbm.at[si_flat.at[pl.ds(0, CHUNK)]], rows.at[0], gsems[0]).start()

    for j in range(PPW // 16):
        pv = ((wid * PPW + j * 16 + iota) * 41) % N_NODES
        di_flat[pl.ds(EPW + j * 16, 16)] = pv

    plsc.subcore_barrier()

    ssems = gsems[NB:]
    gsems = gsems[:NB]

    def _outer(o, carry):
        for b in range(NB):
            ch = o * NB + b
            bg = (b + KLA) % NB
            pltpu.make_async_copy(
                y_hbm.at[si_flat.at[pl.ds(ch * CHUNK, CHUNK)]],
                rows.at[b], gsems[b]).wait()
            pltpu.make_async_copy(
                rows.at[b],
                acc.at[di_flat.at[pl.ds(ch * CHUNK, CHUNK)]],
                ssems[b]).start(add=True)
            nxt = ch + KLA

            @pl.when(nxt < NCH)
            def _():
                @pl.when(nxt >= NB)
                def _():
                    pltpu.make_async_copy(
                        rows.at[bg],
                        acc.at[di_flat.at[pl.ds((nxt - NB) * CHUNK, CHUNK)]],
                        ssems[bg]).wait()
                pltpu.make_async_copy(
                    y_hbm.at[si_flat.at[pl.ds(nxt * CHUNK, CHUNK)]],
                    rows.at[bg], gsems[bg]).start()
        return carry

    lax.fori_loop(0, NCH // NB, _outer, 0)

    for b in range(NB):
        ch = NCH - NB + b
        pltpu.make_async_copy(
            rows.at[b],
            acc.at[di_flat.at[pl.ds(ch * CHUNK, CHUNK)]],
            ssems[b]).wait()

    plsc.subcore_barrier()
    pltpu.sync_copy(acc.at[pl.ds(s * RPT, RPT)],
                    out_hbm.at[pl.ds(s * RPT, RPT), pl.ds(c * HID, HID)])


def _segsum(y, eidx):
    mesh = plsc.VectorSubcoreMesh(core_axis_name="c", subcore_axis_name="s")
    k = functools.partial(
        pl.kernel,
        out_type=jax.ShapeDtypeStruct((N_NODES, 2 * HID), jnp.float32),
        mesh=mesh,
        scratch_types=[
            pltpu.VMEM((NCH * CHUNK,), jnp.int32),
            pltpu.VMEM((NCH * CHUNK,), jnp.int32),
            pltpu.VMEM((NB, CHUNK, HID), jnp.float32),
            pltpu.VMEM_SHARED((N_NODES, HID), jnp.float32),
        ] + [pltpu.SemaphoreType.DMA] * (2 * NB),
        compiler_params=pltpu.CompilerParams(use_tc_tiling_on_sc=False),
    )(_segsum_body)
    return k(y, eidx)


def _finish_body(y_ref, agg_ref, eps_ref, b1_ref, w2_ref, b2_ref,
                 g_ref, be_ref, wl_ref, bl_ref, o_ref):
    agg = agg_ref[:, :HID] + agg_ref[:, HID:]
    h = (1.0 + eps_ref[0, 0]) * y_ref[:N_NODES] + agg + b1_ref[...]
    h = jnp.maximum(h, 0.0)
    h = lax.dot_general(h, w2_ref[...], (((1,), (1,)), ((), ())),
                        preferred_element_type=jnp.float32) + b2_ref[...]
    mean = jnp.mean(h, axis=0, keepdims=True)
    var = jnp.mean((h - mean) ** 2, axis=0, keepdims=True)
    h = (h - mean) * lax.rsqrt(var + 1e-5) * g_ref[...] + be_ref[...]
    h = jnp.maximum(h, 0.0)
    o_ref[...] = lax.dot_general(wl_ref[...], h, (((1,), (1,)), ((), ())),
                                 preferred_element_type=jnp.float32) + bl_ref[...]


def _finish(y, agg2, eps, b1, w2, b2, gamma, beta, wl, bl):
    return pl.pallas_call(
        _finish_body,
        out_shape=jax.ShapeDtypeStruct((HID, N_NODES), jnp.float32),
    )(y, agg2, eps, b1, w2, b2, gamma, beta, wl, bl)


def kernel(x, edge_index, eps, W1, b1, W2, b2, gamma, beta, Wl, bl):
    eidx = edge_index.astype(jnp.int32)
    y = _project(x, W1)
    agg2 = _segsum(y, eidx)
    out_t = _finish(y, agg2,
                    jnp.asarray(eps, jnp.float32).reshape(1, 1),
                    b1.reshape(1, HID), W2, b2.reshape(1, HID),
                    gamma.reshape(1, HID), beta.reshape(1, HID),
                    Wl, bl.reshape(HID, 1))
    return out_t.T


# speedup vs baseline: 1.0335x; 1.0335x over previous
import functools

import jax
import jax.numpy as jnp
from jax import lax
from jax.experimental import pallas as pl
from jax.experimental.pallas import tpu as pltpu
from jax.experimental.pallas import tpu_sc as plsc

N_NODES = 10000
N_EDGES = 320000
D_FEAT = 128
HID = 64

NC = 2
NS = 16
NW = NC * NS
EPW = N_EDGES // NW
CHUNK = 128
NCH = 80
PPW = NCH * CHUNK - EPW
NB = 8
Y_ROWS = N_NODES + 16
RPT = N_NODES // NS


def _proj_body(x_ref, w1_ref, y_ref):
    i = pl.program_id(0)
    y = lax.dot_general(x_ref[...], w1_ref[...], (((1,), (1,)), ((), ())),
                        preferred_element_type=jnp.float32)
    blk = y.shape[0]
    row = i * blk + lax.broadcasted_iota(jnp.int32, y.shape, 0)
    y_ref[...] = jnp.where(row < N_NODES, y, 0.0)


def _project(x, w1):
    blk = 2504
    return pl.pallas_call(
        _proj_body,
        grid=(Y_ROWS // blk,),
        in_specs=[
            pl.BlockSpec((blk, D_FEAT), lambda i: (i, 0)),
            pl.BlockSpec((HID, D_FEAT), lambda i: (0, 0)),
        ],
        out_specs=pl.BlockSpec((blk, HID), lambda i: (i, 0)),
        out_shape=jax.ShapeDtypeStruct((Y_ROWS, HID), jnp.float32),
    )(x, w1)


def _segsum_body(y_hbm, eidx_hbm, out_hbm,
                 si_flat, di_flat, rows, acc, *gsems):
    c = lax.axis_index("c")
    s = lax.axis_index("s")
    wid = s * NC + c
    base = wid * EPW
    iota = lax.iota(jnp.int32, 16)

    pltpu.sync_copy(eidx_hbm.at[0, pl.ds(base, EPW)],
                    si_flat.at[pl.ds(0, EPW)])
    for j in range(PPW // 16):
        si_flat[pl.ds(EPW + j * 16, 16)] = N_NODES + iota
    for b in range(1, NB):
        pltpu.make_async_copy(
            y_hbm.at[si_flat.at[pl.ds(b * CHUNK, CHUNK)]],
            rows.at[b], gsems[b]).start()

    pltpu.sync_copy(eidx_hbm.at[1, pl.ds(base, EPW)],
                    di_flat.at[pl.ds(0, EPW)])

    z16 = jnp.zeros((16,), jnp.float32)

    def _zfill(i, carry):
        rows[0, i // 4, pl.ds((i % 4) * 16, 16)] = z16
        return carry

    lax.fori_loop(0, CHUNK * 4, _zfill, 0)
    for k in range(RPT // CHUNK):
        pltpu.sync_copy(rows.at[0], acc.at[pl.ds(s * RPT + k * CHUNK, CHUNK)])
    _zrem = RPT % CHUNK
    if _zrem:
        pltpu.sync_copy(rows.at[0].at[pl.ds(0, _zrem)],
                        acc.at[pl.ds(s * RPT + (RPT // CHUNK) * CHUNK, _zrem)])
    pltpu.make_async_copy(
        y_hbm.at[si_flat.at[pl.ds(0, CHUNK)]], rows.at[0], gsems[0]).start()

    for j in range(PPW // 16):
        pv = ((wid * PPW + j * 16 + iota) * 41) % N_NODES
        di_flat[pl.ds(EPW + j * 16, 16)] = pv

    plsc.subcore_barrier()

    def _outer(o, carry):
        for b in range(NB):
            ch = o * NB + b
            pltpu.make_async_copy(
                y_hbm.at[si_flat.at[pl.ds(ch * CHUNK, CHUNK)]],
                rows.at[b], gsems[b]).wait()
            pltpu.sync_copy(rows.at[b],
                            acc.at[di_flat.at[pl.ds(ch * CHUNK, CHUNK)]],
                            add=True)

            @pl.when(o < (NCH // NB) - 1)
            def _():
                pltpu.make_async_copy(
                    y_hbm.at[si_flat.at[pl.ds((ch + NB) * CHUNK, CHUNK)]],
                    rows.at[b], gsems[b]).start()
        return carry

    lax.fori_loop(0, NCH // NB, _outer, 0)

    plsc.subcore_barrier()
    pltpu.sync_copy(acc.at[pl.ds(s * RPT, RPT)],
                    out_hbm.at[pl.ds(s * RPT, RPT), pl.ds(c * HID, HID)])


def _segsum(y, eidx):
    mesh = plsc.VectorSubcoreMesh(core_axis_name="c", subcore_axis_name="s")
    k = functools.partial(
        pl.kernel,
        out_type=jax.ShapeDtypeStruct((N_NODES, 2 * HID), jnp.float32),
        mesh=mesh,
        scratch_types=[
            pltpu.VMEM((NCH * CHUNK,), jnp.int32),
            pltpu.VMEM((NCH * CHUNK,), jnp.int32),
            pltpu.VMEM((NB, CHUNK, HID), jnp.float32),
            pltpu.VMEM_SHARED((N_NODES, HID), jnp.float32),
        ] + [pltpu.SemaphoreType.DMA] * NB,
        compiler_params=pltpu.CompilerParams(use_tc_tiling_on_sc=False),
    )(_segsum_body)
    return k(y, eidx)


def _finish_body(y_ref, agg_ref, eps_ref, b1_ref, w2_ref, b2_ref,
                 g_ref, be_ref, wl_ref, bl_ref, o_ref):
    agg = agg_ref[:, :HID] + agg_ref[:, HID:]
    h = (1.0 + eps_ref[0, 0]) * y_ref[:N_NODES] + agg + b1_ref[...]
    h = jnp.maximum(h, 0.0)
    h = lax.dot_general(h, w2_ref[...], (((1,), (1,)), ((), ())),
                        preferred_element_type=jnp.float32) + b2_ref[...]
    mean = jnp.mean(h, axis=0, keepdims=True)
    var = jnp.mean((h - mean) ** 2, axis=0, keepdims=True)
    h = (h - mean) * lax.rsqrt(var + 1e-5) * g_ref[...] + be_ref[...]
    h = jnp.maximum(h, 0.0)
    o_ref[...] = lax.dot_general(wl_ref[...], h, (((1,), (1,)), ((), ())),
                                 preferred_element_type=jnp.float32) + bl_ref[...]


def _finish(y, agg2, eps, b1, w2, b2, gamma, beta, wl, bl):
    return pl.pallas_call(
        _finish_body,
        out_shape=jax.ShapeDtypeStruct((HID, N_NODES), jnp.float32),
    )(y, agg2, eps, b1, w2, b2, gamma, beta, wl, bl)


def kernel(x, edge_index, eps, W1, b1, W2, b2, gamma, beta, Wl, bl):
    eidx = edge_index.astype(jnp.int32)
    y = _project(x, W1)
    agg2 = _segsum(y, eidx)
    out_t = _finish(y, agg2,
                    jnp.asarray(eps, jnp.float32).reshape(1, 1),
                    b1.reshape(1, HID), W2, b2.reshape(1, HID),
                    gamma.reshape(1, HID), beta.reshape(1, HID),
                    Wl, bl.reshape(HID, 1))
    return out_t.T


# speedup vs baseline: 1.0683x; 1.0337x over previous
import functools

import jax
import jax.numpy as jnp
from jax import lax
from jax.experimental import pallas as pl
from jax.experimental.pallas import tpu as pltpu
from jax.experimental.pallas import tpu_sc as plsc

N_NODES = 10000
N_EDGES = 320000
D_FEAT = 128
HID = 64

NC = 2
NS = 16
NW = NC * NS
EPW = N_EDGES // NW
CHUNK = 128
NCH = 80
PPW = NCH * CHUNK - EPW
NB = 8
Y_ROWS = N_NODES + 16
RPT = N_NODES // NS


def _proj_body(x_ref, w1_ref, y_ref):
    i = pl.program_id(0)
    y = lax.dot_general(x_ref[...], w1_ref[...], (((1,), (1,)), ((), ())),
                        preferred_element_type=jnp.float32)
    blk = y.shape[0]
    row = i * blk + lax.broadcasted_iota(jnp.int32, y.shape, 0)
    y_ref[...] = jnp.where(row < N_NODES, y, 0.0)


def _project(x, w1):
    blk = 2504
    return pl.pallas_call(
        _proj_body,
        grid=(Y_ROWS // blk,),
        in_specs=[
            pl.BlockSpec((blk, D_FEAT), lambda i: (i, 0)),
            pl.BlockSpec((HID, D_FEAT), lambda i: (0, 0)),
        ],
        out_specs=pl.BlockSpec((blk, HID), lambda i: (i, 0)),
        out_shape=jax.ShapeDtypeStruct((Y_ROWS, HID), jnp.float32),
    )(x, w1)


def _segsum_body(y_hbm, eidx_hbm, out_hbm,
                 si_flat, di_flat, rows, acc, *gsems):
    c = lax.axis_index("c")
    s = lax.axis_index("s")
    wid = s * NC + c
    base = wid * EPW
    iota = lax.iota(jnp.int32, 16)

    zsem = gsems[NB]
    dsem = gsems[NB + 1]
    gsems = gsems[:NB]

    pltpu.make_async_copy(eidx_hbm.at[0, pl.ds(base, EPW)],
                          si_flat.at[pl.ds(0, EPW)], gsems[0]).start()
    pltpu.make_async_copy(eidx_hbm.at[1, pl.ds(base, EPW)],
                          di_flat.at[pl.ds(0, EPW)], dsem).start()

    z16 = jnp.zeros((16,), jnp.float32)

    def _zfill(i, carry):
        rows[0, i // 4, pl.ds((i % 4) * 16, 16)] = z16
        return carry

    lax.fori_loop(0, CHUNK * 4, _zfill, 0)
    for k in range(RPT // CHUNK):
        pltpu.make_async_copy(
            rows.at[0], acc.at[pl.ds(s * RPT + k * CHUNK, CHUNK)],
            zsem).start()
    _zrem = RPT % CHUNK
    if _zrem:
        pltpu.make_async_copy(
            rows.at[0].at[pl.ds(0, _zrem)],
            acc.at[pl.ds(s * RPT + (RPT // CHUNK) * CHUNK, _zrem)],
            zsem).start()

    pltpu.make_async_copy(eidx_hbm.at[0, pl.ds(base, EPW)],
                          si_flat.at[pl.ds(0, EPW)], gsems[0]).wait()
    for j in range(PPW // 16):
        si_flat[pl.ds(EPW + j * 16, 16)] = N_NODES + iota
    for b in range(1, NB):
        pltpu.make_async_copy(
            y_hbm.at[si_flat.at[pl.ds(b * CHUNK, CHUNK)]],
            rows.at[b], gsems[b]).start()

    for k in range(RPT // CHUNK):
        pltpu.make_async_copy(
            rows.at[0], acc.at[pl.ds(s * RPT + k * CHUNK, CHUNK)],
            zsem).wait()
    if _zrem:
        pltpu.make_async_copy(
            rows.at[0].at[pl.ds(0, _zrem)],
            acc.at[pl.ds(s * RPT + (RPT // CHUNK) * CHUNK, _zrem)],
            zsem).wait()
    pltpu.make_async_copy(
        y_hbm.at[si_flat.at[pl.ds(0, CHUNK)]], rows.at[0], gsems[0]).start()

    pltpu.make_async_copy(eidx_hbm.at[1, pl.ds(base, EPW)],
                          di_flat.at[pl.ds(0, EPW)], dsem).wait()
    for j in range(PPW // 16):
        pv = ((wid * PPW + j * 16 + iota) * 41) % N_NODES
        di_flat[pl.ds(EPW + j * 16, 16)] = pv

    plsc.subcore_barrier()

    def _outer(o, carry):
        for b in range(NB):
            ch = o * NB + b
            pltpu.make_async_copy(
                y_hbm.at[si_flat.at[pl.ds(ch * CHUNK, CHUNK)]],
                rows.at[b], gsems[b]).wait()
            pltpu.sync_copy(rows.at[b],
                            acc.at[di_flat.at[pl.ds(ch * CHUNK, CHUNK)]],
                            add=True)

            @pl.when(o < (NCH // NB) - 1)
            def _():
                pltpu.make_async_copy(
                    y_hbm.at[si_flat.at[pl.ds((ch + NB) * CHUNK, CHUNK)]],
                    rows.at[b], gsems[b]).start()
        return carry

    lax.fori_loop(0, NCH // NB, _outer, 0)

    plsc.subcore_barrier()
    pltpu.sync_copy(acc.at[pl.ds(s * RPT, RPT)],
                    out_hbm.at[pl.ds(s * RPT, RPT), pl.ds(c * HID, HID)])


def _segsum(y, eidx):
    mesh = plsc.VectorSubcoreMesh(core_axis_name="c", subcore_axis_name="s")
    k = functools.partial(
        pl.kernel,
        out_type=jax.ShapeDtypeStruct((N_NODES, 2 * HID), jnp.float32),
        mesh=mesh,
        scratch_types=[
            pltpu.VMEM((NCH * CHUNK,), jnp.int32),
            pltpu.VMEM((NCH * CHUNK,), jnp.int32),
            pltpu.VMEM((NB, CHUNK, HID), jnp.float32),
            pltpu.VMEM_SHARED((N_NODES, HID), jnp.float32),
        ] + [pltpu.SemaphoreType.DMA] * (NB + 2),
        compiler_params=pltpu.CompilerParams(use_tc_tiling_on_sc=False),
    )(_segsum_body)
    return k(y, eidx)


def _finish_body(y_ref, agg_ref, eps_ref, b1_ref, w2_ref, b2_ref,
                 g_ref, be_ref, wl_ref, bl_ref, o_ref):
    agg = agg_ref[:, :HID] + agg_ref[:, HID:]
    h = (1.0 + eps_ref[0, 0]) * y_ref[:N_NODES] + agg + b1_ref[...]
    h = jnp.maximum(h, 0.0)
    h = lax.dot_general(h, w2_ref[...], (((1,), (1,)), ((), ())),
                        preferred_element_type=jnp.float32) + b2_ref[...]
    mean = jnp.mean(h, axis=0, keepdims=True)
    var = jnp.mean((h - mean) ** 2, axis=0, keepdims=True)
    h = (h - mean) * lax.rsqrt(var + 1e-5) * g_ref[...] + be_ref[...]
    h = jnp.maximum(h, 0.0)
    o_ref[...] = lax.dot_general(wl_ref[...], h, (((1,), (1,)), ((), ())),
                                 preferred_element_type=jnp.float32) + bl_ref[...]


def _finish(y, agg2, eps, b1, w2, b2, gamma, beta, wl, bl):
    return pl.pallas_call(
        _finish_body,
        out_shape=jax.ShapeDtypeStruct((HID, N_NODES), jnp.float32),
    )(y, agg2, eps, b1, w2, b2, gamma, beta, wl, bl)


def kernel(x, edge_index, eps, W1, b1, W2, b2, gamma, beta, Wl, bl):
    eidx = edge_index.astype(jnp.int32)
    y = _project(x, W1)
    agg2 = _segsum(y, eidx)
    out_t = _finish(y, agg2,
                    jnp.asarray(eps, jnp.float32).reshape(1, 1),
                    b1.reshape(1, HID), W2, b2.reshape(1, HID),
                    gamma.reshape(1, HID), beta.reshape(1, HID),
                    Wl, bl.reshape(HID, 1))
    return out_t.T
